# baseline (speedup 1.0000x reference)
import jax
import jax.numpy as jnp
from jax import lax
from jax.experimental import pallas as pl
from jax.experimental.pallas import tpu as pltpu
from jax.experimental.pallas import tpu_sc as plsc

B, C, H, W = 16, 96, 128, 128
HW = H * W
K = 1024

NC, NS = 2, 16
NW = NC * NS
L = 16
CAP = 2048


def _norm_body(fm_ref, out_ref):
    def step(c, acc):
        x = fm_ref[0, c]
        return acc + x * x

    x0 = fm_ref[0, 0]
    part = lax.fori_loop(1, 32, step, x0 * x0).reshape(1, H, W)

    @pl.when(pl.program_id(1) == 0)
    def _():
        out_ref[...] = part

    @pl.when(pl.program_id(1) != 0)
    def _():
        out_ref[...] += part


def _norms(fm):
    return pl.pallas_call(
        _norm_body,
        grid=(B, 3),
        in_specs=[pl.BlockSpec((1, 32, H, W), lambda b, c: (b, c, 0, 0))],
        out_specs=pl.BlockSpec((1, H, W), lambda b, c: (b, 0, 0)),
        out_shape=jax.ShapeDtypeStruct((B, H, W), jnp.float32),
    )(fm)


def _topk_body(norms_hbm, idx_hbm, vals_v, hist_v, cv_v, ci_v, cv2_v, ci2_v):
    wid = lax.axis_index("s") * NC + lax.axis_index("c")
    lane = lax.broadcasted_iota(jnp.int32, (L,), 0)
    zero16 = jnp.zeros((L,), jnp.int32)

    @pl.when(wid < B)
    def _():
        pltpu.sync_copy(norms_hbm.at[wid], vals_v)

        def vgroup(j):
            return plsc.bitcast(
                vals_v[j >> 3, pl.ds((j & 7) * L, L)], jnp.uint32)


        def boundary(cum0):
            cum = cum0
            d_star = jnp.int32(0)
            g_new = jnp.int32(0)
            for i in range(15, -1, -1):
                h = hist_v[pl.ds(i * L, L)]
                hr = lax.rev(h, (0,))
                cs = plsc.cumsum(hr)
                cexc = cum + cs - hr
                cond = (cexc < K) & (cexc + hr >= K)
                dvec = (i * L + 15) - lane
                d_star = d_star + jnp.sum(jnp.where(cond, dvec, 0))
                g_new = g_new + jnp.sum(jnp.where(cond, cexc, 0))
                cum = cum + jnp.sum(h)
            return d_star, g_new

        prefix = jnp.uint32(0)
        gt = jnp.int32(0)
        for lvl in range(2):
            shift = 24 - 8 * lvl
            for i in range(16):
                hist_v[pl.ds(i * L, L)] = zero16

            def scan(j, c, shift=shift, prefix=prefix, lvl=lvl):
                v = vgroup(j)
                digit = ((v >> shift) & 0xFF).astype(jnp.int32)
                if lvl == 0:
                    cnt, last = plsc.scan_count(digit)
                else:
                    pred = (v >> (shift + 8)) == prefix
                    cnt, last = plsc.scan_count(digit, pred)
                plsc.addupdate_scatter(hist_v, [digit], cnt, mask=last)
                return c

            lax.fori_loop(0, HW // L, scan, 0, unroll=4)
            d_star, g_new = boundary(gt)
            prefix = (prefix << 8) | d_star.astype(jnp.uint32)
            gt = g_new

        def comp2(j, carry):
            ptr, mq = carry
            v = vgroup(j)
            hi = v >> 16
            idv = j * L + lane
            mgt = hi > prefix
            meq = (hi == prefix) & (mq < CAP - L)
            plsc.store_compressed(
                cv_v.at[pl.ds(ptr, L)], plsc.bitcast(~v, jnp.int32), mask=mgt)
            plsc.store_compressed(ci_v.at[pl.ds(ptr, L)], idv, mask=mgt)
            plsc.store_compressed(
                cv2_v.at[pl.ds(mq, L)], plsc.bitcast(v, jnp.int32), mask=meq)
            plsc.store_compressed(ci2_v.at[pl.ds(mq, L)], idv, mask=meq)
            return (ptr + plsc.all_reduce_population_count(mgt)[0],
                    mq + plsc.all_reduce_population_count(meq)[0])

        gt_n, m = lax.fori_loop(0, HW // L, comp2, (jnp.int32(0), jnp.int32(0)))
        mg = (m + (L - 1)) >> 4

        def egroup(j):
            return plsc.bitcast(cv2_v[pl.ds(j * L, L)], jnp.uint32)

        for lvl in range(2, 4):
            shift = 24 - 8 * lvl
            for i in range(16):
                hist_v[pl.ds(i * L, L)] = zero16

            def scan_eq(j, c, shift=shift, prefix=prefix):
                v = egroup(j)
                valid = (j * L + lane) < m
                pred = valid & ((v >> (shift + 8)) == prefix)
                digit = ((v >> shift) & 0xFF).astype(jnp.int32)
                cnt, last = plsc.scan_count(digit, pred)
                plsc.addupdate_scatter(hist_v, [digit], cnt, mask=last)
                return c

            lax.fori_loop(0, mg, scan_eq, 0)
            d_star, g_new = boundary(gt)
            prefix = (prefix << 8) | d_star.astype(jnp.uint32)
            gt = g_new

        thr = prefix

        def comp3(j, ptr):
            v = egroup(j)
            mm = ((j * L + lane) < m) & (v >= thr) & (ptr < CAP - L)
            plsc.store_compressed(
                cv_v.at[pl.ds(ptr, L)], plsc.bitcast(~v, jnp.int32), mask=mm)
            plsc.store_compressed(
                ci_v.at[pl.ds(ptr, L)], ci2_v[pl.ds(j * L, L)], mask=mm)
            return ptr + plsc.all_reduce_population_count(mm)[0]

        n = lax.fori_loop(0, mg, comp3, gt_n)
        ng = (n + (L - 1)) >> 4

        bufs = [(cv_v, ci_v), (cv2_v, ci2_v)]
        for p in range(4):
            src_v, src_i = bufs[p % 2]
            dst_v, dst_i = bufs[(p + 1) % 2]
            shift = 8 * p
            for i in range(16):
                hist_v[pl.ds(i * L, L)] = zero16

            def cnt_body(j, c, src_v=src_v, shift=shift):
                k = plsc.bitcast(src_v[pl.ds(j * L, L)], jnp.uint32)
                valid = (j * L + lane) < n
                digit = ((k >> shift) & 0xFF).astype(jnp.int32)
                cnt, last = plsc.scan_count(digit, valid)
                plsc.addupdate_scatter(hist_v, [digit], cnt, mask=last)
                return c

            lax.fori_loop(0, ng, cnt_body, 0)

            carry = jnp.int32(0)
            for i in range(16):
                h = hist_v[pl.ds(i * L, L)]
                cs = plsc.cumsum(h)
                hist_v[pl.ds(i * L, L)] = carry + cs - h
                carry = carry + jnp.sum(h)

            def scat(j, c, src_v=src_v, src_i=src_i, dst_v=dst_v,
                     dst_i=dst_i, shift=shift):
                ki = src_v[pl.ds(j * L, L)]
                idv = src_i[pl.ds(j * L, L)]
                valid = (j * L + lane) < n
                digit = ((plsc.bitcast(ki, jnp.uint32) >> shift)
                         & 0xFF).astype(jnp.int32)
                cnt, last = plsc.scan_count(digit, valid)
                base = plsc.load_gather(hist_v, [digit])
                pos = base + cnt - 1
                plsc.store_scatter(dst_v, [pos], ki, mask=valid)
                plsc.store_scatter(dst_i, [pos], idv, mask=valid)
                plsc.addupdate_scatter(hist_v, [digit], cnt, mask=last)
                return c

            lax.fori_loop(0, ng, scat, 0)

        pltpu.sync_copy(ci_v.at[pl.ds(0, K)], idx_hbm.at[wid])


def _topk(norms):
    mesh = plsc.VectorSubcoreMesh(
        core_axis_name="c", subcore_axis_name="s", num_cores=NC, num_subcores=NS)
    f = pl.kernel(
        _topk_body,
        out_type=jax.ShapeDtypeStruct((B, K), jnp.int32),
        mesh=mesh,
        compiler_params=pltpu.CompilerParams(needs_layout_passes=False),
        scratch_types=[
            pltpu.VMEM((H, W), jnp.float32),
            pltpu.VMEM((256,), jnp.int32),
            pltpu.VMEM((CAP,), jnp.int32),
            pltpu.VMEM((CAP,), jnp.int32),
            pltpu.VMEM((CAP,), jnp.int32),
            pltpu.VMEM((CAP,), jnp.int32),
        ],
    )
    return f(norms)


RPW = (B * C) // NW


def _gather_body(fm_hbm, idx_hbm, out_hbm, rA, rB, ih_v, iw_v, sel_v,
                 semA, semB):
    wid = lax.axis_index("s") * NC + lax.axis_index("c")
    base = wid * RPW
    b = base // C
    c0 = base % C

    pltpu.sync_copy(idx_hbm.at[b], iw_v)

    def split(j, _):
        v = iw_v[pl.ds(j * L, L)]
        ih_v[pl.ds(j * L, L)] = v >> 7
        iw_v[pl.ds(j * L, L)] = v & 127
        return 0

    lax.fori_loop(0, K // L, split, 0, unroll=8)

    def gather_row(rbuf, srow):
        def chunk(j, _):
            ih = ih_v[pl.ds(j * L, L)]
            iw = iw_v[pl.ds(j * L, L)]
            sel_v[srow, pl.ds(j * L, L)] = plsc.load_gather(rbuf, [ih, iw])
            return 0
        lax.fori_loop(0, K // L, chunk, 0, unroll=8)

    pltpu.make_async_copy(fm_hbm.at[b, c0], rA, semA).start()
    pltpu.make_async_copy(fm_hbm.at[b, c0 + 1], rB, semB).start()

    def pair(p, _):
        c = c0 + 2 * p
        pltpu.make_async_copy(fm_hbm.at[b, c], rA, semA).wait()
        gather_row(rA, 2 * p)

        @pl.when(2 * p + 2 < RPW)
        def _():
            pltpu.make_async_copy(fm_hbm.at[b, c + 2], rA, semA).start()

        pltpu.make_async_copy(fm_hbm.at[b, c + 1], rB, semB).wait()
        gather_row(rB, 2 * p + 1)

        @pl.when(2 * p + 3 < RPW)
        def _():
            pltpu.make_async_copy(fm_hbm.at[b, c + 3], rB, semB).start()

        return 0

    lax.fori_loop(0, RPW // 2, pair, 0)
    pltpu.sync_copy(sel_v, out_hbm.at[pl.ds(base, RPW)])


def _gather(fm, idx):
    mesh = plsc.VectorSubcoreMesh(
        core_axis_name="c", subcore_axis_name="s", num_cores=NC, num_subcores=NS)
    f = pl.kernel(
        _gather_body,
        out_type=jax.ShapeDtypeStruct((B * C, K), jnp.float32),
        mesh=mesh,
        compiler_params=pltpu.CompilerParams(needs_layout_passes=False),
        scratch_types=[
            pltpu.VMEM((H, W), jnp.float32),
            pltpu.VMEM((H, W), jnp.float32),
            pltpu.VMEM((K,), jnp.int32),
            pltpu.VMEM((K,), jnp.int32),
            pltpu.VMEM((RPW, K), jnp.float32),
            pltpu.SemaphoreType.DMA,
            pltpu.SemaphoreType.DMA,
        ],
    )
    return f(fm, idx)


def kernel(featureMaps):
    fm = featureMaps
    norms = _norms(fm)
    idx = _topk(norms)
    sel = _gather(fm, idx)
    return sel.reshape(B, C, K)


# speedup vs baseline: 1.1778x; 1.1778x over previous
import jax
import jax.numpy as jnp
from jax import lax
from jax.experimental import pallas as pl
from jax.experimental.pallas import tpu as pltpu
from jax.experimental.pallas import tpu_sc as plsc

B, C, H, W = 16, 96, 128, 128
HW = H * W
K = 1024

NC, NS = 2, 16
NW = NC * NS
L = 16
CAP = 3072
NBINS = 2048


def _norm_body(fm_ref, out_ref):
    def fold32(base):
        def step(c, acc):
            x = fm_ref[0, c]
            return acc + x * x
        x0 = fm_ref[0, base]
        return lax.fori_loop(base + 1, base + 32, step, x0 * x0)

    acc = (fold32(0) + fold32(32)) + fold32(64)
    out_ref[...] = acc.reshape(1, H, W)


def _norms(fm):
    return pl.pallas_call(
        _norm_body,
        grid=(B,),
        in_specs=[pl.BlockSpec((1, C, H, W), lambda b: (b, 0, 0, 0))],
        out_specs=pl.BlockSpec((1, H, W), lambda b: (b, 0, 0)),
        out_shape=jax.ShapeDtypeStruct((B, H, W), jnp.float32),
    )(fm)


def _topk_body(norms_hbm, idx_hbm, vals_v, hist_v, cv_v, ci_v, cv2_v, ci2_v):
    wid = lax.axis_index("s") * NC + lax.axis_index("c")
    lane = lax.broadcasted_iota(jnp.int32, (L,), 0)
    zero16 = jnp.zeros((L,), jnp.int32)

    @pl.when(wid < B)
    def _():
        pltpu.sync_copy(norms_hbm.at[wid], vals_v)

        def vgroup(j):
            return plsc.bitcast(
                vals_v[j >> 3, pl.ds((j & 7) * L, L)], jnp.uint32)


        def boundary(cum0, nchunks):
            cum = cum0
            d_star = jnp.int32(0)
            g_new = jnp.int32(0)
            for i in range(nchunks - 1, -1, -1):
                h = hist_v[pl.ds(i * L, L)]
                hr = lax.rev(h, (0,))
                cs = plsc.cumsum(hr)
                cexc = cum + cs - hr
                cond = (cexc < K) & (cexc + hr >= K)
                dvec = (i * L + 15) - lane
                d_star = d_star + jnp.sum(jnp.where(cond, dvec, 0))
                g_new = g_new + jnp.sum(jnp.where(cond, cexc, 0))
                cum = cum + jnp.sum(h)
            return d_star, g_new

        def clear_hist(i, c):
            hist_v[pl.ds(i * L, L)] = zero16
            return c

        lax.fori_loop(0, NBINS // L, clear_hist, 0, unroll=4)

        def scan(j, c):
            v = vgroup(j)
            digit = (v >> 20).astype(jnp.int32)
            cnt, last = plsc.scan_count(digit)
            plsc.addupdate_scatter(hist_v, [digit], cnt, mask=last)
            return c

        lax.fori_loop(0, HW // L, scan, 0, unroll=4)

        def bch(i, carry):
            cum, d_star, g_new = carry
            base = (NBINS // L - 1 - i) * L
            h = hist_v[pl.ds(base, L)]
            hr = lax.rev(h, (0,))
            cs = plsc.cumsum(hr)
            cexc = cum + cs - hr
            cond = (cexc < K) & (cexc + hr >= K)
            dvec = (base + 15) - lane
            d_star = d_star + jnp.sum(jnp.where(cond, dvec, 0))
            g_new = g_new + jnp.sum(jnp.where(cond, cexc, 0))
            return cum + jnp.sum(h), d_star, g_new

        _, d11, gt = lax.fori_loop(
            0, NBINS // L, bch, (jnp.int32(0),) * 3)
        prefix = d11.astype(jnp.uint32)

        def comp2(j, carry):
            ptr, mq = carry
            v = vgroup(j)
            hi = v >> 20
            idv = j * L + lane
            mgt = hi > prefix
            meq = (hi == prefix) & (mq < CAP - L)
            plsc.store_compressed(
                cv_v.at[pl.ds(ptr, L)], plsc.bitcast(~v, jnp.int32), mask=mgt)
            plsc.store_compressed(ci_v.at[pl.ds(ptr, L)], idv, mask=mgt)
            plsc.store_compressed(
                cv2_v.at[pl.ds(mq, L)], plsc.bitcast(v, jnp.int32), mask=meq)
            plsc.store_compressed(ci2_v.at[pl.ds(mq, L)], idv, mask=meq)
            return (ptr + plsc.all_reduce_population_count(mgt)[0],
                    mq + plsc.all_reduce_population_count(meq)[0])

        gt_n, m = lax.fori_loop(0, HW // L, comp2, (jnp.int32(0), jnp.int32(0)))
        mg = (m + (L - 1)) >> 4

        def egroup(j):
            return plsc.bitcast(cv2_v[pl.ds(j * L, L)], jnp.uint32)

        for shift, dmask, pshift in ((12, 0xFF, 20), (4, 0xFF, 12), (0, 0xF, 4)):
            nch = (dmask + 1) // L
            for i in range(nch):
                hist_v[pl.ds(i * L, L)] = zero16

            def scan_eq(j, c, shift=shift, dmask=dmask, pshift=pshift,
                        prefix=prefix):
                v = egroup(j)
                valid = (j * L + lane) < m
                pred = valid & ((v >> pshift) == prefix)
                digit = ((v >> shift) & dmask).astype(jnp.int32)
                cnt, last = plsc.scan_count(digit, pred)
                plsc.addupdate_scatter(hist_v, [digit], cnt, mask=last)
                return c

            lax.fori_loop(0, mg, scan_eq, 0)
            d_star, g_new = boundary(gt, nch)
            prefix = (prefix << (4 if dmask == 0xF else 8)
                      ) | d_star.astype(jnp.uint32)
            gt = g_new

        thr = prefix

        def comp3(j, ptr):
            v = egroup(j)
            mm = ((j * L + lane) < m) & (v >= thr) & (ptr < CAP - L)
            plsc.store_compressed(
                cv_v.at[pl.ds(ptr, L)], plsc.bitcast(~v, jnp.int32), mask=mm)
            plsc.store_compressed(
                ci_v.at[pl.ds(ptr, L)], ci2_v[pl.ds(j * L, L)], mask=mm)
            return ptr + plsc.all_reduce_population_count(mm)[0]

        n = lax.fori_loop(0, mg, comp3, gt_n)
        ng = (n + (L - 1)) >> 4

        bufs = [(cv_v, ci_v), (cv2_v, ci2_v)]
        for p in range(4):
            src_v, src_i = bufs[p % 2]
            dst_v, dst_i = bufs[(p + 1) % 2]
            shift = 8 * p
            for i in range(16):
                hist_v[pl.ds(i * L, L)] = zero16

            def cnt_body(j, c, src_v=src_v, shift=shift):
                k = plsc.bitcast(src_v[pl.ds(j * L, L)], jnp.uint32)
                valid = (j * L + lane) < n
                digit = ((k >> shift) & 0xFF).astype(jnp.int32)
                cnt, last = plsc.scan_count(digit, valid)
                plsc.addupdate_scatter(hist_v, [digit], cnt, mask=last)
                return c

            lax.fori_loop(0, ng, cnt_body, 0)

            carry = jnp.int32(0)
            for i in range(16):
                h = hist_v[pl.ds(i * L, L)]
                cs = plsc.cumsum(h)
                hist_v[pl.ds(i * L, L)] = carry + cs - h
                carry = carry + jnp.sum(h)

            def scat(j, c, src_v=src_v, src_i=src_i, dst_v=dst_v,
                     dst_i=dst_i, shift=shift):
                ki = src_v[pl.ds(j * L, L)]
                idv = src_i[pl.ds(j * L, L)]
                valid = (j * L + lane) < n
                digit = ((plsc.bitcast(ki, jnp.uint32) >> shift)
                         & 0xFF).astype(jnp.int32)
                cnt, last = plsc.scan_count(digit, valid)
                base = plsc.load_gather(hist_v, [digit])
                pos = base + cnt - 1
                plsc.store_scatter(dst_v, [pos], ki, mask=valid)
                plsc.store_scatter(dst_i, [pos], idv, mask=valid)
                plsc.addupdate_scatter(hist_v, [digit], cnt, mask=last)
                return c

            lax.fori_loop(0, ng, scat, 0)

        pltpu.sync_copy(ci_v.at[pl.ds(0, K)], idx_hbm.at[wid])


def _topk(norms):
    mesh = plsc.VectorSubcoreMesh(
        core_axis_name="c", subcore_axis_name="s", num_cores=NC, num_subcores=NS)
    f = pl.kernel(
        _topk_body,
        out_type=jax.ShapeDtypeStruct((B, K), jnp.int32),
        mesh=mesh,
        compiler_params=pltpu.CompilerParams(needs_layout_passes=False),
        scratch_types=[
            pltpu.VMEM((H, W), jnp.float32),
            pltpu.VMEM((NBINS,), jnp.int32),
            pltpu.VMEM((CAP,), jnp.int32),
            pltpu.VMEM((CAP,), jnp.int32),
            pltpu.VMEM((CAP,), jnp.int32),
            pltpu.VMEM((CAP,), jnp.int32),
        ],
    )
    return f(norms)


RPW = (B * C) // NW


def _gather_body(fm_hbm, idx_hbm, out_hbm, rA, rB, ih_v, iw_v, sel_v,
                 semA, semB):
    wid = lax.axis_index("s") * NC + lax.axis_index("c")
    base = wid * RPW
    b = base // C
    c0 = base % C

    pltpu.sync_copy(idx_hbm.at[b], iw_v)

    def split(j, _):
        v = iw_v[pl.ds(j * L, L)]
        ih_v[pl.ds(j * L, L)] = v >> 7
        iw_v[pl.ds(j * L, L)] = v & 127
        return 0

    lax.fori_loop(0, K // L, split, 0, unroll=8)

    def gather_row(rbuf, srow):
        def chunk(j, _):
            ih = ih_v[pl.ds(j * L, L)]
            iw = iw_v[pl.ds(j * L, L)]
            sel_v[srow, pl.ds(j * L, L)] = plsc.load_gather(rbuf, [ih, iw])
            return 0
        lax.fori_loop(0, K // L, chunk, 0, unroll=8)

    pltpu.make_async_copy(fm_hbm.at[b, c0], rA, semA).start()
    pltpu.make_async_copy(fm_hbm.at[b, c0 + 1], rB, semB).start()

    def pair(p, _):
        c = c0 + 2 * p
        pltpu.make_async_copy(fm_hbm.at[b, c], rA, semA).wait()
        gather_row(rA, 2 * p)

        @pl.when(2 * p + 2 < RPW)
        def _():
            pltpu.make_async_copy(fm_hbm.at[b, c + 2], rA, semA).start()

        pltpu.make_async_copy(fm_hbm.at[b, c + 1], rB, semB).wait()
        gather_row(rB, 2 * p + 1)

        @pl.when(2 * p + 3 < RPW)
        def _():
            pltpu.make_async_copy(fm_hbm.at[b, c + 3], rB, semB).start()

        return 0

    lax.fori_loop(0, RPW // 2, pair, 0)
    pltpu.sync_copy(sel_v, out_hbm.at[pl.ds(base, RPW)])


def _gather(fm, idx):
    mesh = plsc.VectorSubcoreMesh(
        core_axis_name="c", subcore_axis_name="s", num_cores=NC, num_subcores=NS)
    f = pl.kernel(
        _gather_body,
        out_type=jax.ShapeDtypeStruct((B * C, K), jnp.float32),
        mesh=mesh,
        compiler_params=pltpu.CompilerParams(needs_layout_passes=False),
        scratch_types=[
            pltpu.VMEM((H, W), jnp.float32),
            pltpu.VMEM((H, W), jnp.float32),
            pltpu.VMEM((K,), jnp.int32),
            pltpu.VMEM((K,), jnp.int32),
            pltpu.VMEM((RPW, K), jnp.float32),
            pltpu.SemaphoreType.DMA,
            pltpu.SemaphoreType.DMA,
        ],
    )
    return f(fm, idx)


def kernel(featureMaps):
    fm = featureMaps
    norms = _norms(fm)
    idx = _topk(norms)
    sel = _gather(fm, idx)
    return sel.reshape(B, C, K)
